# baseline (speedup 1.0000x reference)
import functools

import jax
import jax.numpy as jnp
from jax import lax
from jax.experimental import pallas as pl
from jax.experimental.pallas import tpu as pltpu
from jax.experimental.pallas import tpu_sc as plsc

EMB_DIM = 16
LANES = 128
CHUNK_PROWS = 200


@functools.cache
def _build_broadcast(n_edges: int, emb_dim: int):
    info = plsc.get_sparse_core_info()
    num_workers = info.num_cores * info.num_subcores
    rows_fold = LANES // emb_dim
    assert n_edges % (rows_fold * CHUNK_PROWS) == 0
    n_prows = n_edges // rows_fold
    n_chunks = n_prows // CHUNK_PROWS

    mesh = plsc.VectorSubcoreMesh(core_axis_name="c", subcore_axis_name="s")

    @functools.partial(
        pl.kernel,
        mesh=mesh,
        out_type=jax.ShapeDtypeStruct((n_prows, LANES), jnp.float32),
        scratch_types=[
            pltpu.VMEM((CHUNK_PROWS, LANES), jnp.float32),
            pltpu.SemaphoreType.DMA,
        ],
    )
    def bcast(w_hbm, out_hbm, buf, sem):
        wid = lax.axis_index("s") * info.num_cores + lax.axis_index("c")
        for j in range(rows_fold):
            pltpu.sync_copy(w_hbm.at[0], buf.at[0, pl.ds(j * emb_dim, emb_dim)])
        row0 = [
            buf[0, pl.ds(j * emb_dim, emb_dim)] for j in range(rows_fold)
        ]

        def fill(i, _):
            for j in range(rows_fold):
                buf[i, pl.ds(j * emb_dim, emb_dim)] = row0[j]
            return 0

        lax.fori_loop(1, CHUNK_PROWS, fill, 0)

        n_mine = (n_chunks - 1 - wid) // num_workers + 1

        def start(i, _):
            base = (wid + i * num_workers) * CHUNK_PROWS
            pltpu.make_async_copy(
                buf, out_hbm.at[pl.ds(base, CHUNK_PROWS)], sem
            ).start()
            return 0

        def drain(i, _):
            pltpu.make_async_copy(
                buf, out_hbm.at[pl.ds(wid * CHUNK_PROWS, CHUNK_PROWS)], sem
            ).wait()
            return 0

        lax.fori_loop(0, n_mine, start, 0)
        lax.fori_loop(0, n_mine, drain, 0)

    return bcast


def kernel(batch, W):
    n_edges = batch.shape[0]
    folded = _build_broadcast(n_edges, EMB_DIM)(W)
    return folded.reshape(n_edges, EMB_DIM)


# speedup vs baseline: 14.4951x; 14.4951x over previous
import functools

import jax
import jax.numpy as jnp
from jax import lax
from jax.experimental import pallas as pl
from jax.experimental.pallas import tpu as pltpu
from jax.experimental.pallas import tpu_sc as plsc

EMB_DIM = 16
CHUNK_COLS = 3200


@functools.cache
def _build_broadcast(n_edges: int, emb_dim: int):
    info = plsc.get_sparse_core_info()
    num_workers = info.num_cores * info.num_subcores
    lanes = info.num_lanes
    assert n_edges % CHUNK_COLS == 0
    n_chunks = n_edges // CHUNK_COLS

    mesh = plsc.VectorSubcoreMesh(core_axis_name="c", subcore_axis_name="s")

    @functools.partial(
        pl.kernel,
        mesh=mesh,
        out_type=jax.ShapeDtypeStruct((emb_dim, n_edges), jnp.float32),
        scratch_types=[
            pltpu.VMEM((2 * emb_dim,), jnp.float32),
            pltpu.VMEM((emb_dim, CHUNK_COLS), jnp.float32),
            pltpu.SemaphoreType.DMA,
        ],
    )
    def bcast(w_hbm, out_hbm, wv, buf, sem):
        wid = lax.axis_index("s") * info.num_cores + lax.axis_index("c")
        pltpu.sync_copy(w_hbm.at[0], wv.at[pl.ds(0, emb_dim)])
        pltpu.sync_copy(w_hbm.at[0], wv.at[pl.ds(emb_dim, emb_dim)])
        splats = [
            jnp.full((lanes,), wv[pl.ds(c, lanes)][0], jnp.float32)
            for c in range(emb_dim)
        ]

        def fill(j, _):
            for c in range(emb_dim):
                buf[c, pl.ds(j * lanes, lanes)] = splats[c]
            return 0

        lax.fori_loop(0, CHUNK_COLS // lanes, fill, 0)

        n_mine = (n_chunks - 1 - wid) // num_workers + 1

        def start(i, _):
            base = (wid + i * num_workers) * CHUNK_COLS
            pltpu.make_async_copy(
                buf, out_hbm.at[:, pl.ds(base, CHUNK_COLS)], sem
            ).start()
            return 0

        def drain(i, _):
            pltpu.make_async_copy(
                buf, out_hbm.at[:, pl.ds(wid * CHUNK_COLS, CHUNK_COLS)], sem
            ).wait()
            return 0

        lax.fori_loop(0, n_mine, start, 0)
        lax.fori_loop(0, n_mine, drain, 0)

    return bcast


def kernel(batch, W):
    n_edges = batch.shape[0]
    cols = _build_broadcast(n_edges, EMB_DIM)(W)
    return cols.T
